# baseline (speedup 1.0000x reference)
import functools

import jax
import jax.numpy as jnp
from jax import lax
from jax.experimental import pallas as pl
from jax.experimental.pallas import tpu as pltpu
from jax.experimental.pallas import tpu_sc as plsc

B, S, D = 4, 2048, 768
E, K, DFF = 64, 2, 256
N = B * S
P2 = N * K
C = int(N * K * 2.0 / E)
SENT = 1 << 30

NW = 32
EPT = E // NW
SPT = EPT * C
TPT = N // NW


def _router_body(x_ref, rw_ref, b_ref, logits_ref, e_ref, w_ref, x16_ref):
    x = x_ref[...]
    x16_ref[...] = x.astype(jnp.bfloat16)
    logits = lax.dot_general(x, rw_ref[...], (((1,), (1,)), ((), ())),
                             preferred_element_type=jnp.float32)
    logits = logits + b_ref[...]
    logits_ref[...] = logits

    m = jnp.max(logits, axis=-1, keepdims=True)
    ex = jnp.exp(logits - m)
    probs = ex / jnp.sum(ex, axis=-1, keepdims=True)

    ids = lax.broadcasted_iota(jnp.int32, probs.shape, 1)
    m1 = jnp.max(probs, axis=-1, keepdims=True)
    i1 = jnp.min(jnp.where(probs == m1, ids, E), axis=-1, keepdims=True)
    probs2 = jnp.where(ids == i1, -jnp.inf, probs)
    m2 = jnp.max(probs2, axis=-1, keepdims=True)
    i2 = jnp.min(jnp.where(probs2 == m2, ids, E), axis=-1, keepdims=True)
    denom = m1 + m2 + 1e-9
    e_ref[0, :] = i1[:, 0]
    e_ref[1, :] = i2[:, 0]
    w_ref[0, :] = (m1 / denom)[:, 0]
    w_ref[1, :] = (m2 / denom)[:, 0]


def _router(x2, router_w, bias):
    TB = 2048
    grid = (N // TB,)
    return pl.pallas_call(
        _router_body,
        grid=grid,
        in_specs=[
            pl.BlockSpec((TB, D), lambda i: (i, 0)),
            pl.BlockSpec((E, D), lambda i: (0, 0)),
            pl.BlockSpec((1, E), lambda i: (0, 0)),
        ],
        out_specs=[
            pl.BlockSpec((TB, E), lambda i: (i, 0)),
            pl.BlockSpec((2, TB), lambda i: (0, i)),
            pl.BlockSpec((2, TB), lambda i: (0, i)),
            pl.BlockSpec((TB, D), lambda i: (i, 0)),
        ],
        out_shape=[
            jax.ShapeDtypeStruct((N, E), jnp.float32),
            jax.ShapeDtypeStruct((2, N), jnp.int32),
            jax.ShapeDtypeStruct((2, N), jnp.float32),
            jax.ShapeDtypeStruct((N, D), jnp.bfloat16),
        ],
    )(x2, router_w, bias)


def _shared_body(x_ref, wg_ref, wu_ref, wd_ref, shared_ref):
    x = x_ref[...].astype(jnp.bfloat16)
    wg = wg_ref[...].astype(jnp.bfloat16)
    wu = wu_ref[...].astype(jnp.bfloat16)
    wd = wd_ref[...].astype(jnp.bfloat16)
    g = lax.dot_general(x, wg, (((1,), (1,)), ((), ())),
                        preferred_element_type=jnp.float32)
    u = lax.dot_general(x, wu, (((1,), (1,)), ((), ())),
                        preferred_element_type=jnp.float32)
    a = ((g * jax.nn.sigmoid(g)) * u).astype(jnp.bfloat16)
    shared_ref[...] = 0.5 * lax.dot_general(
        a, wd, (((1,), (1,)), ((), ())),
        preferred_element_type=jnp.float32)


def _shared_ffn(x2, wg_cat, wu_cat, wd_cat):
    TB = 1024
    return pl.pallas_call(
        _shared_body,
        grid=(N // TB,),
        in_specs=[
            pl.BlockSpec((TB, D), lambda i: (i, 0)),
            pl.BlockSpec((2 * DFF, D), lambda i: (0, 0)),
            pl.BlockSpec((2 * DFF, D), lambda i: (0, 0)),
            pl.BlockSpec((D, 2 * DFF), lambda i: (0, 0)),
        ],
        out_specs=pl.BlockSpec((TB, D), lambda i: (i, 0)),
        out_shape=jax.ShapeDtypeStruct((N, D), jnp.float32),
    )(x2, wg_cat, wu_cat, wd_cat)


def _pos_body(e_ref, slot_ref, cnt_ref):
    lanes = lax.broadcasted_iota(jnp.int32, (N, E), 1)

    def excl_cumsum(a):
        G, R = 64, N // 64
        rr = lax.broadcasted_iota(jnp.int32, (R, R), 0)
        cc = lax.broadcasted_iota(jnp.int32, (R, R), 1)
        tri = (cc <= rr).astype(jnp.float32)
        trib = jnp.broadcast_to(tri, (G, R, R))
        ar = a.reshape(G, R, E)
        incl = lax.dot_general(trib, ar, (((2,), (1,)), ((0,), (0,))),
                               preferred_element_type=jnp.float32)
        tot = incl[:, R - 1, :]
        gg = lax.broadcasted_iota(jnp.int32, (G, G), 0)
        hh = lax.broadcasted_iota(jnp.int32, (G, G), 1)
        trig = (hh <= gg).astype(jnp.float32)
        gincl = lax.dot_general(trig, tot, (((1,), (0,)), ((), ())),
                                preferred_element_type=jnp.float32)
        full = incl + (gincl - tot)[:, None, :]
        return full.reshape(N, E) - a, gincl[G - 1, :]

    e1 = e_ref[0, :].reshape(N, 1)
    e2 = e_ref[1, :].reshape(N, 1)
    a1 = (lanes == e1).astype(jnp.int32)
    a2 = (lanes == e2).astype(jnp.int32)
    stf, tf = excl_cumsum((a1 + a2).astype(jnp.float32))
    st = stf.astype(jnp.int32)
    t = tf
    sel = jnp.sum(st * (a1 + (a2 << 16)), axis=1)
    pos1 = sel & 0xFFFF
    pos2 = sel >> 16
    slot_ref[0, :] = jnp.where(pos1 < C, e1[:, 0] * C + pos1, SENT)
    slot_ref[1, :] = jnp.where(pos2 < C, e2[:, 0] * C + pos2, SENT)
    cnt_ref[...] = t.astype(jnp.float32).reshape(1, E)


def _positions(e2):
    return pl.pallas_call(
        _pos_body,
        out_shape=[
            jax.ShapeDtypeStruct((2, N), jnp.int32),
            jax.ShapeDtypeStruct((1, E), jnp.float32),
        ],
    )(e2)


_GCH = 64
_NBUF = 2


PPT = P2 // NW
_NDCH = PPT // _GCH


def _dispatch_body(x_hbm, slot_hbm, ein_hbm, idx_v,
                   row0_v, row1_v, gsem0, gsem1, wsem0, wsem1):
    wid = lax.axis_index("s") * 2 + lax.axis_index("c")
    pbase = wid * PPT
    tbase = pbase & (N - 1)

    for ch in range(_NDCH):
        pltpu.sync_copy(
            slot_hbm.at[pl.ds(pl.multiple_of(pbase + ch * _GCH, _GCH), _GCH)],
            idx_v.at[ch * 8])
    dummy = jnp.full((16,), E * C, jnp.int32) + wid
    for ch in range(_NDCH):
        for j in range(_GCH // 16):
            sv = idx_v[ch * 8, pl.ds(j * 16, 16)]
            idx_v[ch * 8, pl.ds(j * 16, 16)] = jnp.where(sv < E * C, sv, dummy)

    rows = (row0_v, row1_v)
    gsems = (gsem0, gsem1)
    wsems = (wsem0, wsem1)

    def read_desc(ch):
        b = ch % _NBUF
        return pltpu.make_async_copy(
            x_hbm.at[pl.ds(pl.multiple_of(tbase + ch * _GCH, _GCH), _GCH)],
            rows[b], gsems[b])

    def scat_desc(ch):
        b = ch % _NBUF
        return pltpu.make_async_copy(
            rows[b], ein_hbm.at[idx_v.at[ch * 8]], wsems[b])

    for ch in range(_NBUF - 1):
        read_desc(ch).start()
    for ch in range(_NDCH):
        read_desc(ch).wait()
        scat_desc(ch).start()
        if ch + _NBUF - 1 < _NDCH:
            if ch >= 1:
                scat_desc(ch - 1).wait()
            read_desc(ch + _NBUF - 1).start()
    for ch in range(max(0, _NDCH - _NBUF), _NDCH):
        scat_desc(ch).wait()


def _dispatch(x16, slot3):
    mesh = plsc.VectorSubcoreMesh(core_axis_name="c", subcore_axis_name="s")
    return pl.kernel(
        _dispatch_body,
        out_type=jax.ShapeDtypeStruct((E * C + NW, D // 2), jnp.int32),
        mesh=mesh,
        compiler_params=pltpu.CompilerParams(needs_layout_passes=False),
        scratch_types=(
            [pltpu.VMEM((_NDCH * 8, _GCH), jnp.int32)]
            + [pltpu.VMEM((_GCH, D // 2), jnp.int32)] * _NBUF
            + [pltpu.SemaphoreType.DMA] * (2 * _NBUF)
        ),
    )(x16, slot3)


def _ffn_body(x_ref, wg_ref, wu_ref, wd_ref, o_ref):
    xe = x_ref[...]
    wg = wg_ref[0].astype(jnp.bfloat16)
    wu = wu_ref[0].astype(jnp.bfloat16)
    wd = wd_ref[0].astype(jnp.bfloat16)
    g = lax.dot_general(xe, wg, (((1,), (1,)), ((), ())),
                        preferred_element_type=jnp.float32)
    u = lax.dot_general(xe, wu, (((1,), (1,)), ((), ())),
                        preferred_element_type=jnp.float32)
    a = ((g * jax.nn.sigmoid(g)) * u).astype(jnp.bfloat16)
    o_ref[0] = lax.dot_general(a, wd, (((1,), (1,)), ((), ())),
                               preferred_element_type=jnp.float32)


def _expert_ffn(ein, sp_wg, sp_wu, sp_wd):
    return pl.pallas_call(
        _ffn_body,
        grid=(E,),
        in_specs=[
            pl.BlockSpec((C, D), lambda e: (e, 0)),
            pl.BlockSpec((1, DFF, D), lambda e: (e, 0, 0)),
            pl.BlockSpec((1, DFF, D), lambda e: (e, 0, 0)),
            pl.BlockSpec((1, D, DFF), lambda e: (e, 0, 0)),
        ],
        out_specs=pl.BlockSpec((1, C, D), lambda e: (e, 0, 0)),
        out_shape=jax.ShapeDtypeStruct((E, C, D), jnp.float32),
    )(ein, sp_wg, sp_wu, sp_wd)


_CCH = 16
_NV = D // 16


def _combine_body(eo_hbm, slot_hbm, w_hbm, shared_hbm, out_hbm,
                  idx1_v, idx2_v, w1_v, w2_v,
                  r1a, r1b, r2a, r2b, sha, shb, outa, outb,
                  isem0, isem1, osem0, osem1):
    wid = lax.axis_index("s") * 2 + lax.axis_index("c")
    base = wid * TPT

    pltpu.sync_copy(slot_hbm.at[pl.ds(base, TPT)], idx1_v)
    pltpu.sync_copy(slot_hbm.at[pl.ds(N + base, TPT)], idx2_v)
    pltpu.sync_copy(w_hbm.at[pl.ds(base, TPT)], w1_v)
    pltpu.sync_copy(w_hbm.at[pl.ds(N + base, TPT)], w2_v)

    def clean(i, _):
        sl = pl.ds(pl.multiple_of(i * 16, 16), 16)
        s1 = idx1_v[sl]
        s2 = idx2_v[sl]
        v1 = s1 < E * C
        v2 = s2 < E * C
        idx1_v[sl] = jnp.where(v1, s1, 0)
        idx2_v[sl] = jnp.where(v2, s2, 0)
        w1_v[sl] = jnp.where(v1, w1_v[sl], 0.0)
        w2_v[sl] = jnp.where(v2, w2_v[sl], 0.0)
        return 0
    lax.fori_loop(0, TPT // 16, clean, 0)

    r1 = (r1a, r1b)
    r2 = (r2a, r2b)
    sh = (sha, shb)
    outv = (outa, outb)
    isems = (isem0, isem1)
    osems = (osem0, osem1)
    NCH = TPT // _CCH

    def in_descs(ch):
        b = ch & 1
        off = ch * _CCH
        return (
            pltpu.make_async_copy(eo_hbm.at[idx1_v.at[pl.ds(off, _CCH)]],
                                  r1[b], isems[b]),
            pltpu.make_async_copy(eo_hbm.at[idx2_v.at[pl.ds(off, _CCH)]],
                                  r2[b], isems[b]),
            pltpu.make_async_copy(shared_hbm.at[pl.ds(base + off, _CCH)],
                                  sh[b], isems[b]),
        )

    def out_desc(ch):
        b = ch & 1
        return pltpu.make_async_copy(
            outv[b], out_hbm.at[pl.ds(base + ch * _CCH, _CCH)], osems[b])

    for ch in range(NCH):
        b = ch & 1
        if ch == 0:
            for dsc in in_descs(0):
                dsc.start()
        if ch + 1 < NCH:
            for dsc in in_descs(ch + 1):
                dsc.start()
        if ch >= 2:
            out_desc(ch - 2).wait()
        for dsc in in_descs(ch):
            dsc.wait()
        off = ch * _CCH

        def row(t, _):
            idxv = jnp.full((16,), off + t, jnp.int32)
            w1 = plsc.load_gather(w1_v, [idxv])
            w2 = plsc.load_gather(w2_v, [idxv])
            v1 = w1 > 0.0
            v2 = w2 > 0.0
            zero = jnp.zeros((16,), jnp.float32)
            for v in range(_NV):
                sl = pl.ds(v * 16, 16)
                t1 = jnp.where(v1, w1 * r1[b][t, sl], zero)
                t2 = jnp.where(v2, w2 * r2[b][t, sl], zero)
                outv[b][t, sl] = sh[b][t, sl] + t1 + t2
            return 0
        lax.fori_loop(0, _CCH, row, 0)
        out_desc(ch).start()
    out_desc(NCH - 2).wait()
    out_desc(NCH - 1).wait()


def _combine(eo_flat, slot_flat, w_flat, shared):
    mesh = plsc.VectorSubcoreMesh(core_axis_name="c", subcore_axis_name="s")
    return pl.kernel(
        _combine_body,
        out_type=jax.ShapeDtypeStruct((N, D), jnp.float32),
        mesh=mesh,
        compiler_params=pltpu.CompilerParams(needs_layout_passes=False),
        scratch_types=[
            pltpu.VMEM((TPT,), jnp.int32),
            pltpu.VMEM((TPT,), jnp.int32),
            pltpu.VMEM((TPT,), jnp.float32),
            pltpu.VMEM((TPT,), jnp.float32),
            pltpu.VMEM((_CCH, D), jnp.float32),
            pltpu.VMEM((_CCH, D), jnp.float32),
            pltpu.VMEM((_CCH, D), jnp.float32),
            pltpu.VMEM((_CCH, D), jnp.float32),
            pltpu.VMEM((_CCH, D), jnp.float32),
            pltpu.VMEM((_CCH, D), jnp.float32),
            pltpu.VMEM((_CCH, D), jnp.float32),
            pltpu.VMEM((_CCH, D), jnp.float32),
            pltpu.SemaphoreType.DMA,
            pltpu.SemaphoreType.DMA,
            pltpu.SemaphoreType.DMA,
            pltpu.SemaphoreType.DMA,
        ],
    )(eo_flat, slot_flat, w_flat, shared)


@jax.jit
def kernel(x, router_w, bias, shared_wg, shared_wu, shared_wd,
           sp_wg, sp_wu, sp_wd):
    x2 = x.reshape(N, D)
    wg_cat = shared_wg.reshape(2 * DFF, D)
    wu_cat = shared_wu.reshape(2 * DFF, D)
    wd_cat = jnp.concatenate([shared_wd[0], shared_wd[1]], axis=1)

    logits, e2, w2, x16 = _router(x2, router_w, bias)
    slots, counts = _positions(e2)
    slot_flat = slots.reshape(P2)
    x16i = lax.bitcast_convert_type(x16.reshape(N, D // 2, 2), jnp.int32)
    ein_i = _dispatch(x16i, slot_flat)
    shared = _shared_ffn(x2, wg_cat, wu_cat, wd_cat)
    ein = lax.bitcast_convert_type(ein_i, jnp.bfloat16).reshape(E * C + NW, D)
    eout = _expert_ffn(ein, sp_wg, sp_wu, sp_wd)
    out = _combine(eout.reshape(E * C, D), slot_flat, w2.reshape(P2), shared)

    return (out.reshape(B, S, D), logits.reshape(B, S, E), counts.reshape(E))


# speedup vs baseline: 3.2860x; 3.2860x over previous
import functools

import jax
import jax.numpy as jnp
from jax import lax
from jax.experimental import pallas as pl
from jax.experimental.pallas import tpu as pltpu
from jax.experimental.pallas import tpu_sc as plsc

B, S, D = 4, 2048, 768
E, K, DFF = 64, 2, 256
N = B * S
P2 = N * K
C = int(N * K * 2.0 / E)
SENT = 1 << 30

NW = 32
EPT = E // NW
SPT = EPT * C
TPT = N // NW


def _router_body(x_ref, rw_ref, b_ref, logits_ref, e_ref, w_ref, x16_ref):
    x = x_ref[...]
    b = pltpu.bitcast(x, jnp.int32)
    r = b + 0x7FFF + ((b >> 16) & 1)
    lo = lax.shift_right_logical(r[:, : D // 2], 16)
    hi = r[:, D // 2:] & jnp.int32(-65536)
    x16_ref[...] = lo | hi
    logits = lax.dot_general(x, rw_ref[...], (((1,), (1,)), ((), ())),
                             preferred_element_type=jnp.float32)
    logits = logits + b_ref[...]
    logits_ref[...] = logits

    m = jnp.max(logits, axis=-1, keepdims=True)
    ex = jnp.exp(logits - m)
    probs = ex / jnp.sum(ex, axis=-1, keepdims=True)

    ids = lax.broadcasted_iota(jnp.int32, probs.shape, 1)
    m1 = jnp.max(probs, axis=-1, keepdims=True)
    i1 = jnp.min(jnp.where(probs == m1, ids, E), axis=-1, keepdims=True)
    probs2 = jnp.where(ids == i1, -jnp.inf, probs)
    m2 = jnp.max(probs2, axis=-1, keepdims=True)
    i2 = jnp.min(jnp.where(probs2 == m2, ids, E), axis=-1, keepdims=True)
    denom = m1 + m2 + 1e-9
    e_ref[0, :] = i1[:, 0]
    e_ref[1, :] = i2[:, 0]
    w_ref[0, :] = (m1 / denom)[:, 0]
    w_ref[1, :] = (m2 / denom)[:, 0]


def _router(x2, router_w, bias):
    TB = 2048
    grid = (N // TB,)
    return pl.pallas_call(
        _router_body,
        grid=grid,
        in_specs=[
            pl.BlockSpec((TB, D), lambda i: (i, 0)),
            pl.BlockSpec((E, D), lambda i: (0, 0)),
            pl.BlockSpec((1, E), lambda i: (0, 0)),
        ],
        out_specs=[
            pl.BlockSpec((TB, E), lambda i: (i, 0)),
            pl.BlockSpec((2, TB), lambda i: (0, i)),
            pl.BlockSpec((2, TB), lambda i: (0, i)),
            pl.BlockSpec((TB, D // 2), lambda i: (i, 0)),
        ],
        out_shape=[
            jax.ShapeDtypeStruct((N, E), jnp.float32),
            jax.ShapeDtypeStruct((2, N), jnp.int32),
            jax.ShapeDtypeStruct((2, N), jnp.float32),
            jax.ShapeDtypeStruct((N, D // 2), jnp.int32),
        ],
    )(x2, router_w, bias)


def _shared_body(x_ref, wg_ref, wu_ref, wd_ref, shared_ref):
    x = x_ref[...].astype(jnp.bfloat16)
    wg = wg_ref[...].astype(jnp.bfloat16)
    wu = wu_ref[...].astype(jnp.bfloat16)
    wd = wd_ref[...].astype(jnp.bfloat16)
    g = lax.dot_general(x, wg, (((1,), (1,)), ((), ())),
                        preferred_element_type=jnp.float32)
    u = lax.dot_general(x, wu, (((1,), (1,)), ((), ())),
                        preferred_element_type=jnp.float32)
    a = ((g * jax.nn.sigmoid(g)) * u).astype(jnp.bfloat16)
    shared_ref[...] = 0.5 * lax.dot_general(
        a, wd, (((1,), (1,)), ((), ())),
        preferred_element_type=jnp.float32)


def _shared_ffn(x2, wg_cat, wu_cat, wd_cat):
    TB = 1024
    return pl.pallas_call(
        _shared_body,
        grid=(N // TB,),
        in_specs=[
            pl.BlockSpec((TB, D), lambda i: (i, 0)),
            pl.BlockSpec((2 * DFF, D), lambda i: (0, 0)),
            pl.BlockSpec((2 * DFF, D), lambda i: (0, 0)),
            pl.BlockSpec((D, 2 * DFF), lambda i: (0, 0)),
        ],
        out_specs=pl.BlockSpec((TB, D), lambda i: (i, 0)),
        out_shape=jax.ShapeDtypeStruct((N, D), jnp.float32),
    )(x2, wg_cat, wu_cat, wd_cat)


def _pos_body(e_ref, slot_ref, cnt_ref):
    lanes = lax.broadcasted_iota(jnp.int32, (N, E), 1)

    def excl_cumsum(a):
        G, R = 64, N // 64
        rr = lax.broadcasted_iota(jnp.int32, (R, R), 0)
        cc = lax.broadcasted_iota(jnp.int32, (R, R), 1)
        tri = (cc <= rr).astype(jnp.float32)
        trib = jnp.broadcast_to(tri, (G, R, R))
        ar = a.reshape(G, R, E)
        incl = lax.dot_general(trib, ar, (((2,), (1,)), ((0,), (0,))),
                               preferred_element_type=jnp.float32)
        tot = incl[:, R - 1, :]
        gg = lax.broadcasted_iota(jnp.int32, (G, G), 0)
        hh = lax.broadcasted_iota(jnp.int32, (G, G), 1)
        trig = (hh <= gg).astype(jnp.float32)
        gincl = lax.dot_general(trig, tot, (((1,), (0,)), ((), ())),
                                preferred_element_type=jnp.float32)
        full = incl + (gincl - tot)[:, None, :]
        return full.reshape(N, E) - a, gincl[G - 1, :]

    e1 = e_ref[0, :].reshape(N, 1)
    e2 = e_ref[1, :].reshape(N, 1)
    a1 = (lanes == e1).astype(jnp.int32)
    a2 = (lanes == e2).astype(jnp.int32)
    stf, tf = excl_cumsum((a1 + a2).astype(jnp.float32))
    st = stf.astype(jnp.int32)
    t = tf
    sel = jnp.sum(st * (a1 + (a2 << 16)), axis=1)
    pos1 = sel & 0xFFFF
    pos2 = sel >> 16
    slot_ref[0, :] = jnp.where(pos1 < C, e1[:, 0] * C + pos1, SENT)
    slot_ref[1, :] = jnp.where(pos2 < C, e2[:, 0] * C + pos2, SENT)
    cnt_ref[...] = t.astype(jnp.float32).reshape(1, E)


def _positions(e2):
    return pl.pallas_call(
        _pos_body,
        out_shape=[
            jax.ShapeDtypeStruct((2, N), jnp.int32),
            jax.ShapeDtypeStruct((1, E), jnp.float32),
        ],
    )(e2)


_GCH = 64
_NBUF = 2


PPT = P2 // NW
_NDCH = PPT // _GCH


def _dispatch_body(x_hbm, slot_hbm, ein_hbm, idx_v,
                   row0_v, row1_v, gsem0, gsem1, wsem0, wsem1):
    wid = lax.axis_index("s") * 2 + lax.axis_index("c")
    pbase = wid * PPT
    tbase = pbase & (N - 1)

    for ch in range(_NDCH):
        pltpu.sync_copy(
            slot_hbm.at[pl.ds(pl.multiple_of(pbase + ch * _GCH, _GCH), _GCH)],
            idx_v.at[ch * 8])
    dummy = jnp.full((16,), E * C, jnp.int32) + wid
    for ch in range(_NDCH):
        for j in range(_GCH // 16):
            sv = idx_v[ch * 8, pl.ds(j * 16, 16)]
            idx_v[ch * 8, pl.ds(j * 16, 16)] = jnp.where(sv < E * C, sv, dummy)

    rows = (row0_v, row1_v)
    gsems = (gsem0, gsem1)
    wsems = (wsem0, wsem1)

    def read_desc(ch):
        b = ch % _NBUF
        return pltpu.make_async_copy(
            x_hbm.at[pl.ds(pl.multiple_of(tbase + ch * _GCH, _GCH), _GCH)],
            rows[b], gsems[b])

    def scat_desc(ch):
        b = ch % _NBUF
        return pltpu.make_async_copy(
            rows[b], ein_hbm.at[idx_v.at[ch * 8]], wsems[b])

    for ch in range(_NBUF - 1):
        read_desc(ch).start()
    for ch in range(_NDCH):
        read_desc(ch).wait()
        scat_desc(ch).start()
        if ch + _NBUF - 1 < _NDCH:
            if ch >= 1:
                scat_desc(ch - 1).wait()
            read_desc(ch + _NBUF - 1).start()
    for ch in range(max(0, _NDCH - _NBUF), _NDCH):
        scat_desc(ch).wait()


def _dispatch(x16, slot3):
    mesh = plsc.VectorSubcoreMesh(core_axis_name="c", subcore_axis_name="s")
    return pl.kernel(
        _dispatch_body,
        out_type=jax.ShapeDtypeStruct((E * C + NW, D // 2), jnp.int32),
        mesh=mesh,
        compiler_params=pltpu.CompilerParams(needs_layout_passes=False),
        scratch_types=(
            [pltpu.VMEM((_NDCH * 8, _GCH), jnp.int32)]
            + [pltpu.VMEM((_GCH, D // 2), jnp.int32)] * _NBUF
            + [pltpu.SemaphoreType.DMA] * (2 * _NBUF)
        ),
    )(x16, slot3)


def _ffn_body(x_ref, wg_ref, wu_ref, wd_ref, o_ref):
    xi = x_ref[...]
    xlo = pltpu.bitcast(xi << 16, jnp.float32)
    xhi = pltpu.bitcast(xi & jnp.int32(-65536), jnp.float32)
    xe = jnp.concatenate([xlo, xhi], axis=1)
    g = lax.dot_general(xe, wg_ref[0], (((1,), (1,)), ((), ())),
                        preferred_element_type=jnp.float32)
    u = lax.dot_general(xe, wu_ref[0], (((1,), (1,)), ((), ())),
                        preferred_element_type=jnp.float32)
    a = (g * jax.nn.sigmoid(g)) * u
    o_ref[0] = lax.dot_general(a, wd_ref[0], (((1,), (1,)), ((), ())),
                               preferred_element_type=jnp.float32)


def _expert_ffn(ein, sp_wg, sp_wu, sp_wd):
    return pl.pallas_call(
        _ffn_body,
        grid=(E,),
        in_specs=[
            pl.BlockSpec((C, D // 2), lambda e: (e, 0)),
            pl.BlockSpec((1, DFF, D), lambda e: (e, 0, 0)),
            pl.BlockSpec((1, DFF, D), lambda e: (e, 0, 0)),
            pl.BlockSpec((1, D, DFF), lambda e: (e, 0, 0)),
        ],
        out_specs=pl.BlockSpec((1, C, D), lambda e: (e, 0, 0)),
        out_shape=jax.ShapeDtypeStruct((E, C, D), jnp.float32),
    )(ein, sp_wg, sp_wu, sp_wd)


_CCH = 16
_NV = D // 16


def _combine_body(eo_hbm, slot_hbm, w_hbm, shared_hbm, out_hbm,
                  idx1_v, idx2_v, w1_v, w2_v,
                  r1a, r1b, r2a, r2b, sha, shb, outa, outb,
                  isem0, isem1, osem0, osem1):
    wid = lax.axis_index("s") * 2 + lax.axis_index("c")
    base = wid * TPT

    pltpu.sync_copy(slot_hbm.at[pl.ds(base, TPT)], idx1_v)
    pltpu.sync_copy(slot_hbm.at[pl.ds(N + base, TPT)], idx2_v)
    pltpu.sync_copy(w_hbm.at[pl.ds(base, TPT)], w1_v)
    pltpu.sync_copy(w_hbm.at[pl.ds(N + base, TPT)], w2_v)

    def clean(i, _):
        sl = pl.ds(pl.multiple_of(i * 16, 16), 16)
        s1 = idx1_v[sl]
        s2 = idx2_v[sl]
        v1 = s1 < E * C
        v2 = s2 < E * C
        idx1_v[sl] = jnp.where(v1, s1, 0)
        idx2_v[sl] = jnp.where(v2, s2, 0)
        w1_v[sl] = jnp.where(v1, w1_v[sl], 0.0)
        w2_v[sl] = jnp.where(v2, w2_v[sl], 0.0)
        return 0
    lax.fori_loop(0, TPT // 16, clean, 0)

    r1 = (r1a, r1b)
    r2 = (r2a, r2b)
    sh = (sha, shb)
    outv = (outa, outb)
    isems = (isem0, isem1)
    osems = (osem0, osem1)
    NCH = TPT // _CCH

    def in_descs(ch):
        b = ch & 1
        off = ch * _CCH
        return (
            pltpu.make_async_copy(eo_hbm.at[idx1_v.at[pl.ds(off, _CCH)]],
                                  r1[b], isems[b]),
            pltpu.make_async_copy(eo_hbm.at[idx2_v.at[pl.ds(off, _CCH)]],
                                  r2[b], isems[b]),
            pltpu.make_async_copy(shared_hbm.at[pl.ds(base + off, _CCH)],
                                  sh[b], isems[b]),
        )

    def out_desc(ch):
        b = ch & 1
        return pltpu.make_async_copy(
            outv[b], out_hbm.at[pl.ds(base + ch * _CCH, _CCH)], osems[b])

    for ch in range(NCH):
        b = ch & 1
        if ch == 0:
            for dsc in in_descs(0):
                dsc.start()
        if ch + 1 < NCH:
            for dsc in in_descs(ch + 1):
                dsc.start()
        if ch >= 2:
            out_desc(ch - 2).wait()
        for dsc in in_descs(ch):
            dsc.wait()
        off = ch * _CCH

        def row(t, _):
            idxv = jnp.full((16,), off + t, jnp.int32)
            w1 = plsc.load_gather(w1_v, [idxv])
            w2 = plsc.load_gather(w2_v, [idxv])
            v1 = w1 > 0.0
            v2 = w2 > 0.0
            zero = jnp.zeros((16,), jnp.float32)
            for v in range(_NV):
                sl = pl.ds(v * 16, 16)
                t1 = jnp.where(v1, w1 * r1[b][t, sl], zero)
                t2 = jnp.where(v2, w2 * r2[b][t, sl], zero)
                outv[b][t, sl] = sh[b][t, sl] + t1 + t2
            return 0
        lax.fori_loop(0, _CCH, row, 0)
        out_desc(ch).start()
    out_desc(NCH - 2).wait()
    out_desc(NCH - 1).wait()


def _combine(eo_flat, slot_flat, w_flat, shared):
    mesh = plsc.VectorSubcoreMesh(core_axis_name="c", subcore_axis_name="s")
    return pl.kernel(
        _combine_body,
        out_type=jax.ShapeDtypeStruct((N, D), jnp.float32),
        mesh=mesh,
        compiler_params=pltpu.CompilerParams(needs_layout_passes=False),
        scratch_types=[
            pltpu.VMEM((TPT,), jnp.int32),
            pltpu.VMEM((TPT,), jnp.int32),
            pltpu.VMEM((TPT,), jnp.float32),
            pltpu.VMEM((TPT,), jnp.float32),
            pltpu.VMEM((_CCH, D), jnp.float32),
            pltpu.VMEM((_CCH, D), jnp.float32),
            pltpu.VMEM((_CCH, D), jnp.float32),
            pltpu.VMEM((_CCH, D), jnp.float32),
            pltpu.VMEM((_CCH, D), jnp.float32),
            pltpu.VMEM((_CCH, D), jnp.float32),
            pltpu.VMEM((_CCH, D), jnp.float32),
            pltpu.VMEM((_CCH, D), jnp.float32),
            pltpu.SemaphoreType.DMA,
            pltpu.SemaphoreType.DMA,
            pltpu.SemaphoreType.DMA,
            pltpu.SemaphoreType.DMA,
        ],
    )(eo_flat, slot_flat, w_flat, shared)


@jax.jit
def kernel(x, router_w, bias, shared_wg, shared_wu, shared_wd,
           sp_wg, sp_wu, sp_wd):
    x2 = x.reshape(N, D)
    wg_cat = shared_wg.reshape(2 * DFF, D)
    wu_cat = shared_wu.reshape(2 * DFF, D)
    wd_cat = jnp.concatenate([shared_wd[0], shared_wd[1]], axis=1)

    logits, e2, w2, x16 = _router(x2, router_w, bias)
    slots, counts = _positions(e2)
    slot_flat = slots.reshape(P2)
    ein = _dispatch(x16, slot_flat)
    shared = _shared_ffn(x2, wg_cat, wu_cat, wd_cat)
    eout = _expert_ffn(ein, sp_wg, sp_wu, sp_wd)
    out = _combine(eout.reshape(E * C, D), slot_flat, w2.reshape(P2), shared)

    return (out.reshape(B, S, D), logits.reshape(B, S, E), counts.reshape(E))


# speedup vs baseline: 3.3265x; 1.0123x over previous
import functools

import jax
import jax.numpy as jnp
from jax import lax
from jax.experimental import pallas as pl
from jax.experimental.pallas import tpu as pltpu
from jax.experimental.pallas import tpu_sc as plsc

B, S, D = 4, 2048, 768
E, K, DFF = 64, 2, 256
N = B * S
P2 = N * K
C = int(N * K * 2.0 / E)
SENT = 1 << 30

NW = 32
EPT = E // NW
SPT = EPT * C
TPT = N // NW


def _router_body(x_ref, rw_ref, b_ref, logits_ref, e_ref, w_ref, x16_ref):
    x = x_ref[...]
    b = pltpu.bitcast(x, jnp.int32)
    r = b + 0x7FFF + ((b >> 16) & 1)
    lo = lax.shift_right_logical(r[:, : D // 2], 16)
    hi = r[:, D // 2:] & jnp.int32(-65536)
    x16_ref[...] = lo | hi
    logits = lax.dot_general(x, rw_ref[...], (((1,), (1,)), ((), ())),
                             preferred_element_type=jnp.float32)
    logits = logits + b_ref[...]
    logits_ref[...] = logits

    m = jnp.max(logits, axis=-1, keepdims=True)
    ex = jnp.exp(logits - m)
    probs = ex / jnp.sum(ex, axis=-1, keepdims=True)

    ids = lax.broadcasted_iota(jnp.int32, probs.shape, 1)
    m1 = jnp.max(probs, axis=-1, keepdims=True)
    i1 = jnp.min(jnp.where(probs == m1, ids, E), axis=-1, keepdims=True)
    probs2 = jnp.where(ids == i1, -jnp.inf, probs)
    m2 = jnp.max(probs2, axis=-1, keepdims=True)
    i2 = jnp.min(jnp.where(probs2 == m2, ids, E), axis=-1, keepdims=True)
    denom = m1 + m2 + 1e-9
    e_ref[0, :] = i1[:, 0]
    e_ref[1, :] = i2[:, 0]
    w_ref[0, :] = (m1 / denom)[:, 0]
    w_ref[1, :] = (m2 / denom)[:, 0]


def _router(x2, router_w, bias):
    TB = 2048
    grid = (N // TB,)
    return pl.pallas_call(
        _router_body,
        grid=grid,
        in_specs=[
            pl.BlockSpec((TB, D), lambda i: (i, 0)),
            pl.BlockSpec((E, D), lambda i: (0, 0)),
            pl.BlockSpec((1, E), lambda i: (0, 0)),
        ],
        out_specs=[
            pl.BlockSpec((TB, E), lambda i: (i, 0)),
            pl.BlockSpec((2, TB), lambda i: (0, i)),
            pl.BlockSpec((2, TB), lambda i: (0, i)),
            pl.BlockSpec((TB, D // 2), lambda i: (i, 0)),
        ],
        out_shape=[
            jax.ShapeDtypeStruct((N, E), jnp.float32),
            jax.ShapeDtypeStruct((2, N), jnp.int32),
            jax.ShapeDtypeStruct((2, N), jnp.float32),
            jax.ShapeDtypeStruct((N, D // 2), jnp.int32),
        ],
    )(x2, router_w, bias)


def _shared_body(x_ref, wg_ref, wu_ref, wd_ref, shared_ref):
    x = x_ref[...].astype(jnp.bfloat16)
    wg = wg_ref[...].astype(jnp.bfloat16)
    wu = wu_ref[...].astype(jnp.bfloat16)
    wd = wd_ref[...].astype(jnp.bfloat16)
    g = lax.dot_general(x, wg, (((1,), (1,)), ((), ())),
                        preferred_element_type=jnp.float32)
    u = lax.dot_general(x, wu, (((1,), (1,)), ((), ())),
                        preferred_element_type=jnp.float32)
    a = ((g * jax.nn.sigmoid(g)) * u).astype(jnp.bfloat16)
    shared_ref[...] = 0.5 * lax.dot_general(
        a, wd, (((1,), (1,)), ((), ())),
        preferred_element_type=jnp.float32)


def _shared_ffn(x2, wg_cat, wu_cat, wd_cat):
    TB = 1024
    return pl.pallas_call(
        _shared_body,
        grid=(N // TB,),
        in_specs=[
            pl.BlockSpec((TB, D), lambda i: (i, 0)),
            pl.BlockSpec((2 * DFF, D), lambda i: (0, 0)),
            pl.BlockSpec((2 * DFF, D), lambda i: (0, 0)),
            pl.BlockSpec((D, 2 * DFF), lambda i: (0, 0)),
        ],
        out_specs=pl.BlockSpec((TB, D), lambda i: (i, 0)),
        out_shape=jax.ShapeDtypeStruct((N, D), jnp.float32),
    )(x2, wg_cat, wu_cat, wd_cat)


def _pos_body(e_ref, slot_ref, cnt_ref):
    lanes = lax.broadcasted_iota(jnp.int32, (N, E), 1)

    def excl_cumsum(a):
        G, R = 64, N // 64
        rr = lax.broadcasted_iota(jnp.int32, (R, R), 0)
        cc = lax.broadcasted_iota(jnp.int32, (R, R), 1)
        tri = (cc <= rr).astype(jnp.float32)
        trib = jnp.broadcast_to(tri, (G, R, R))
        ar = a.reshape(G, R, E)
        incl = lax.dot_general(trib, ar, (((2,), (1,)), ((0,), (0,))),
                               preferred_element_type=jnp.float32)
        tot = incl[:, R - 1, :]
        gg = lax.broadcasted_iota(jnp.int32, (G, G), 0)
        hh = lax.broadcasted_iota(jnp.int32, (G, G), 1)
        trig = (hh <= gg).astype(jnp.float32)
        gincl = lax.dot_general(trig, tot, (((1,), (0,)), ((), ())),
                                preferred_element_type=jnp.float32)
        full = incl + (gincl - tot)[:, None, :]
        return full.reshape(N, E) - a, gincl[G - 1, :]

    e1 = e_ref[0, :].reshape(N, 1)
    e2 = e_ref[1, :].reshape(N, 1)
    a1 = (lanes == e1).astype(jnp.int32)
    a2 = (lanes == e2).astype(jnp.int32)
    stf, tf = excl_cumsum((a1 + a2).astype(jnp.float32))
    st = stf.astype(jnp.int32)
    t = tf
    sel = jnp.sum(st * (a1 + (a2 << 16)), axis=1)
    pos1 = sel & 0xFFFF
    pos2 = sel >> 16
    slot_ref[0, :] = jnp.where(pos1 < C, e1[:, 0] * C + pos1, SENT)
    slot_ref[1, :] = jnp.where(pos2 < C, e2[:, 0] * C + pos2, SENT)
    cnt_ref[...] = t.astype(jnp.float32).reshape(1, E)


def _positions(e2):
    return pl.pallas_call(
        _pos_body,
        out_shape=[
            jax.ShapeDtypeStruct((2, N), jnp.int32),
            jax.ShapeDtypeStruct((1, E), jnp.float32),
        ],
    )(e2)


_GCH = 64
_NBUF = 2


PPT = P2 // NW
_NDCH = PPT // _GCH


def _dispatch_body(x_hbm, slot_hbm, ein_hbm, idx_v,
                   row0_v, row1_v, gsem0, gsem1, wsem0, wsem1):
    wid = lax.axis_index("s") * 2 + lax.axis_index("c")
    pbase = wid * PPT
    tbase = pbase & (N - 1)

    for ch in range(_NDCH):
        pltpu.sync_copy(
            slot_hbm.at[pl.ds(pl.multiple_of(pbase + ch * _GCH, _GCH), _GCH)],
            idx_v.at[ch * 8])
    dummy = jnp.full((16,), E * C, jnp.int32) + wid
    for ch in range(_NDCH):
        for j in range(_GCH // 16):
            sv = idx_v[ch * 8, pl.ds(j * 16, 16)]
            idx_v[ch * 8, pl.ds(j * 16, 16)] = jnp.where(sv < E * C, sv, dummy)

    rows = (row0_v, row1_v)
    gsems = (gsem0, gsem1)
    wsems = (wsem0, wsem1)

    def read_desc(ch):
        b = ch % _NBUF
        return pltpu.make_async_copy(
            x_hbm.at[pl.ds(pl.multiple_of(tbase + ch * _GCH, _GCH), _GCH)],
            rows[b], gsems[b])

    def scat_desc(ch):
        b = ch % _NBUF
        return pltpu.make_async_copy(
            rows[b], ein_hbm.at[idx_v.at[ch * 8]], wsems[b])

    for ch in range(_NBUF - 1):
        read_desc(ch).start()
    for ch in range(_NDCH):
        read_desc(ch).wait()
        scat_desc(ch).start()
        if ch + _NBUF - 1 < _NDCH:
            if ch >= 1:
                scat_desc(ch - 1).wait()
            read_desc(ch + _NBUF - 1).start()
    for ch in range(max(0, _NDCH - _NBUF), _NDCH):
        scat_desc(ch).wait()


def _dispatch(x16, slot3):
    mesh = plsc.VectorSubcoreMesh(core_axis_name="c", subcore_axis_name="s")
    return pl.kernel(
        _dispatch_body,
        out_type=jax.ShapeDtypeStruct((E * C + NW, D // 2), jnp.int32),
        mesh=mesh,
        compiler_params=pltpu.CompilerParams(needs_layout_passes=False),
        scratch_types=(
            [pltpu.VMEM((_NDCH * 8, _GCH), jnp.int32)]
            + [pltpu.VMEM((_GCH, D // 2), jnp.int32)] * _NBUF
            + [pltpu.SemaphoreType.DMA] * (2 * _NBUF)
        ),
    )(x16, slot3)


def _ffn_body(x_ref, wg_ref, wu_ref, wd_ref, o_ref):
    xi = x_ref[...]
    xlo = pltpu.bitcast(xi << 16, jnp.float32)
    xhi = pltpu.bitcast(xi & jnp.int32(-65536), jnp.float32)
    xe = jnp.concatenate([xlo, xhi], axis=1)
    g = lax.dot_general(xe, wg_ref[0], (((1,), (1,)), ((), ())),
                        preferred_element_type=jnp.float32)
    u = lax.dot_general(xe, wu_ref[0], (((1,), (1,)), ((), ())),
                        preferred_element_type=jnp.float32)
    a = (g * jax.nn.sigmoid(g)) * u
    o = lax.dot_general(a, wd_ref[0], (((1,), (1,)), ((), ())),
                        preferred_element_type=jnp.float32)
    bo = pltpu.bitcast(o, jnp.int32)
    ro = bo + 0x7FFF + ((bo >> 16) & 1)
    o_ref[0] = (lax.shift_right_logical(ro[:, : D // 2], 16)
                | (ro[:, D // 2:] & jnp.int32(-65536)))


def _expert_ffn(ein, sp_wg, sp_wu, sp_wd):
    return pl.pallas_call(
        _ffn_body,
        grid=(E,),
        in_specs=[
            pl.BlockSpec((C, D // 2), lambda e: (e, 0)),
            pl.BlockSpec((1, DFF, D), lambda e: (e, 0, 0)),
            pl.BlockSpec((1, DFF, D), lambda e: (e, 0, 0)),
            pl.BlockSpec((1, D, DFF), lambda e: (e, 0, 0)),
        ],
        out_specs=pl.BlockSpec((1, C, D // 2), lambda e: (e, 0, 0)),
        out_shape=jax.ShapeDtypeStruct((E, C, D // 2), jnp.int32),
    )(ein, sp_wg, sp_wu, sp_wd)


_CCH = 16
_NV = D // 16


def _combine_body(eo_hbm, slot_hbm, w_hbm, shared_hbm, out_hbm,
                  idx1_v, idx2_v, w1_v, w2_v,
                  r1a, r1b, r2a, r2b, sha, shb, outa, outb,
                  isem0, isem1, osem0, osem1):
    wid = lax.axis_index("s") * 2 + lax.axis_index("c")
    base = wid * TPT

    pltpu.sync_copy(slot_hbm.at[pl.ds(base, TPT)], idx1_v)
    pltpu.sync_copy(slot_hbm.at[pl.ds(N + base, TPT)], idx2_v)
    pltpu.sync_copy(w_hbm.at[pl.ds(base, TPT)], w1_v)
    pltpu.sync_copy(w_hbm.at[pl.ds(N + base, TPT)], w2_v)

    def clean(i, _):
        sl = pl.ds(pl.multiple_of(i * 16, 16), 16)
        s1 = idx1_v[sl]
        s2 = idx2_v[sl]
        v1 = s1 < E * C
        v2 = s2 < E * C
        idx1_v[sl] = jnp.where(v1, s1, 0)
        idx2_v[sl] = jnp.where(v2, s2, 0)
        w1_v[sl] = jnp.where(v1, w1_v[sl], 0.0)
        w2_v[sl] = jnp.where(v2, w2_v[sl], 0.0)
        return 0
    lax.fori_loop(0, TPT // 16, clean, 0)

    r1 = (r1a, r1b)
    r2 = (r2a, r2b)
    sh = (sha, shb)
    outv = (outa, outb)
    isems = (isem0, isem1)
    osems = (osem0, osem1)
    NCH = TPT // _CCH

    def in_descs(ch, b):
        off = pl.multiple_of(ch * _CCH, _CCH)
        return (
            pltpu.make_async_copy(eo_hbm.at[idx1_v.at[pl.ds(off, _CCH)]],
                                  r1[b], isems[b]),
            pltpu.make_async_copy(eo_hbm.at[idx2_v.at[pl.ds(off, _CCH)]],
                                  r2[b], isems[b]),
            pltpu.make_async_copy(shared_hbm.at[pl.ds(base + off, _CCH)],
                                  sh[b], isems[b]),
        )

    def out_desc(ch, b):
        off = pl.multiple_of(ch * _CCH, _CCH)
        return pltpu.make_async_copy(
            outv[b], out_hbm.at[pl.ds(base + off, _CCH)], osems[b])

    def do_chunk(ch, b):
        for dsc in in_descs(ch, b):
            dsc.wait()
        off = pl.multiple_of(ch * _CCH, _CCH)

        def row(t, _):
            idxv = jnp.full((16,), off + t, jnp.int32)
            w1 = plsc.load_gather(w1_v, [idxv])
            w2 = plsc.load_gather(w2_v, [idxv])
            v1 = w1 > 0.0
            v2 = w2 > 0.0
            zero = jnp.zeros((16,), jnp.float32)
            himask = jnp.full((16,), -65536, jnp.int32)
            for v in range(_NV // 2):
                sl = pl.ds(v * 16, 16)
                sh2 = pl.ds(D // 2 + v * 16, 16)
                x1 = r1[b][t, sl]
                x2 = r2[b][t, sl]
                lo1 = plsc.bitcast(x1 << 16, jnp.float32)
                hi1 = plsc.bitcast(x1 & himask, jnp.float32)
                lo2 = plsc.bitcast(x2 << 16, jnp.float32)
                hi2 = plsc.bitcast(x2 & himask, jnp.float32)
                outv[b][t, sl] = (sh[b][t, sl]
                                  + jnp.where(v1, w1 * lo1, zero)
                                  + jnp.where(v2, w2 * lo2, zero))
                outv[b][t, sh2] = (sh[b][t, sh2]
                                   + jnp.where(v1, w1 * hi1, zero)
                                   + jnp.where(v2, w2 * hi2, zero))
            return 0
        lax.fori_loop(0, _CCH, row, 0)
        out_desc(ch, b).start()

    for dsc in in_descs(0, 0):
        dsc.start()

    def pair(cp, _):
        ch0 = cp * 2
        for dsc in in_descs(ch0 + 1, 1):
            dsc.start()
        pl.when(cp >= 1)(lambda: out_desc(ch0 - 2, 0).wait())
        do_chunk(ch0, 0)
        pl.when(cp + 1 < NCH // 2)(
            lambda: [d.start() for d in in_descs(ch0 + 2, 0)] and None)
        pl.when(cp >= 1)(lambda: out_desc(ch0 - 1, 1).wait())
        do_chunk(ch0 + 1, 1)
        return 0
    lax.fori_loop(0, NCH // 2, pair, 0)
    out_desc(NCH - 2, 0).wait()
    out_desc(NCH - 1, 1).wait()


def _combine(eo_flat, slot_flat, w_flat, shared):
    mesh = plsc.VectorSubcoreMesh(core_axis_name="c", subcore_axis_name="s")
    return pl.kernel(
        _combine_body,
        out_type=jax.ShapeDtypeStruct((N, D), jnp.float32),
        mesh=mesh,
        compiler_params=pltpu.CompilerParams(needs_layout_passes=False),
        scratch_types=[
            pltpu.VMEM((TPT,), jnp.int32),
            pltpu.VMEM((TPT,), jnp.int32),
            pltpu.VMEM((TPT,), jnp.float32),
            pltpu.VMEM((TPT,), jnp.float32),
            pltpu.VMEM((_CCH, D // 2), jnp.int32),
            pltpu.VMEM((_CCH, D // 2), jnp.int32),
            pltpu.VMEM((_CCH, D // 2), jnp.int32),
            pltpu.VMEM((_CCH, D // 2), jnp.int32),
            pltpu.VMEM((_CCH, D), jnp.float32),
            pltpu.VMEM((_CCH, D), jnp.float32),
            pltpu.VMEM((_CCH, D), jnp.float32),
            pltpu.VMEM((_CCH, D), jnp.float32),
            pltpu.SemaphoreType.DMA,
            pltpu.SemaphoreType.DMA,
            pltpu.SemaphoreType.DMA,
            pltpu.SemaphoreType.DMA,
        ],
    )(eo_flat, slot_flat, w_flat, shared)


@jax.jit
def kernel(x, router_w, bias, shared_wg, shared_wu, shared_wd,
           sp_wg, sp_wu, sp_wd):
    x2 = x.reshape(N, D)
    wg_cat = shared_wg.reshape(2 * DFF, D)
    wu_cat = shared_wu.reshape(2 * DFF, D)
    wd_cat = jnp.concatenate([shared_wd[0], shared_wd[1]], axis=1)

    logits, e2, w2, x16 = _router(x2, router_w, bias)
    slots, counts = _positions(e2)
    slot_flat = slots.reshape(P2)
    ein = _dispatch(x16, slot_flat)
    shared = _shared_ffn(x2, wg_cat, wu_cat, wd_cat)
    eout = _expert_ffn(ein, sp_wg, sp_wu, sp_wd)
    out = _combine(eout.reshape(E * C, D // 2), slot_flat, w2.reshape(P2),
                   shared)

    return (out.reshape(B, S, D), logits.reshape(B, S, E), counts.reshape(E))
